# Initial kernel scaffold; baseline (speedup 1.0000x reference)
import jax
import jax.numpy as jnp
from jax.experimental import pallas as pl


def kernel(test_feature_ids, feature_vectors):
    raise NotImplementedError("write your pallas kernel here")


# speedup vs baseline: 1.5620x; 1.5620x over previous
import jax
import jax.numpy as jnp
from jax import lax
from jax.experimental import pallas as pl
from jax.experimental.pallas import tpu as pltpu
from jax.experimental.pallas import tpu_sc as plsc

BATCH = 16384
FIELDS = 26
NUM_FEATURES = 32
TOTAL = BATCH * FIELDS

NUM_CORES = 2
NUM_SUBCORES = 16
NW = NUM_CORES * NUM_SUBCORES
B_PER_W = TOTAL // NW
CHUNK = 1664
STEPS = B_PER_W // CHUNK


def _gather_body(idx_hbm, table_hbm, out_hbm, idx_v, rows_v, sem):
    c = lax.axis_index("c")
    s = lax.axis_index("s")
    wid = s * NUM_CORES + c
    base = wid * B_PER_W
    for i in range(STEPS):
        off = base + i * CHUNK
        pltpu.sync_copy(idx_hbm.at[pl.ds(off, CHUNK)], idx_v)
        pltpu.async_copy(table_hbm.at[idx_v], rows_v, sem).wait()
        pltpu.sync_copy(rows_v, out_hbm.at[pl.ds(off, CHUNK)])


@jax.jit
def kernel(test_feature_ids, feature_vectors):
    flat_idx = test_feature_ids.reshape(TOTAL)
    gathered = pl.kernel(
        _gather_body,
        out_type=jax.ShapeDtypeStruct((TOTAL, NUM_FEATURES), jnp.float32),
        mesh=plsc.VectorSubcoreMesh(core_axis_name="c", subcore_axis_name="s"),
        scratch_types=[
            pltpu.VMEM((CHUNK,), jnp.int32),
            pltpu.VMEM((CHUNK, NUM_FEATURES), jnp.float32),
            pltpu.SemaphoreType.DMA,
        ],
        compiler_params=pltpu.CompilerParams(use_tc_tiling_on_sc=False),
    )(flat_idx, feature_vectors)
    return gathered.reshape(BATCH, FIELDS, NUM_FEATURES)


# speedup vs baseline: 1.5763x; 1.0092x over previous
import jax
import jax.numpy as jnp
from jax import lax
from jax.experimental import pallas as pl
from jax.experimental.pallas import tpu as pltpu
from jax.experimental.pallas import tpu_sc as plsc

BATCH = 16384
FIELDS = 26
NUM_FEATURES = 32
TOTAL = BATCH * FIELDS

NUM_CORES = 2
NUM_SUBCORES = 16
NW = NUM_CORES * NUM_SUBCORES
B_PER_W = TOTAL // NW
CHUNK = 1024
STEPS = B_PER_W // CHUNK
NBUF = 3


def _gather_body(idx_hbm, table_hbm, out_hbm, idx_v, rows_v, gsems, wsems):
    c = lax.axis_index("c")
    s = lax.axis_index("s")
    wid = s * NUM_CORES + c
    base = wid * B_PER_W

    pltpu.sync_copy(idx_hbm.at[pl.ds(base, B_PER_W)], idx_v)

    def gather(step, buf):
        return pltpu.async_copy(
            table_hbm.at[idx_v.at[pl.ds(step * CHUNK, CHUNK)]],
            rows_v.at[buf],
            gsems[buf],
        )

    gathers = {}
    writes = {}
    for b in range(NBUF):
        gathers[b] = gather(b, b)
    for i in range(STEPS):
        b = i % NBUF
        gathers.pop(b).wait()
        writes[b] = pltpu.async_copy(
            rows_v.at[b], out_hbm.at[pl.ds(base + i * CHUNK, CHUNK)], wsems[b]
        )
        nxt = i + NBUF
        if nxt < STEPS:
            writes.pop(b).wait()
            gathers[b] = gather(nxt, b)
    for b, w in writes.items():
        w.wait()


@jax.jit
def kernel(test_feature_ids, feature_vectors):
    flat_idx = test_feature_ids.reshape(TOTAL)
    gathered = pl.kernel(
        _gather_body,
        out_type=jax.ShapeDtypeStruct((TOTAL, NUM_FEATURES), jnp.float32),
        mesh=plsc.VectorSubcoreMesh(core_axis_name="c", subcore_axis_name="s"),
        scratch_types=[
            pltpu.VMEM((B_PER_W,), jnp.int32),
            pltpu.VMEM((NBUF, CHUNK, NUM_FEATURES), jnp.float32),
            [pltpu.SemaphoreType.DMA] * NBUF,
            [pltpu.SemaphoreType.DMA] * NBUF,
        ],
        compiler_params=pltpu.CompilerParams(use_tc_tiling_on_sc=False),
    )(flat_idx, feature_vectors)
    return gathered.reshape(BATCH, FIELDS, NUM_FEATURES)
